# Initial kernel scaffold; baseline (speedup 1.0000x reference)
import functools

import jax
import jax.numpy as jnp
from jax import lax
from jax.experimental import pallas as pl
from jax.experimental.pallas import tpu as pltpu
from jax.experimental.pallas import tpu_sc as plsc

B, S, D = 1024, 20, 64
H = 4
DH = D // H
USR = 8
GLOB = D - USR
POOL = 10000

NROWS = B * S
PPAD = 10240
PCHUNK = 1024
NCHUNK = PPAD // PCHUNK
RBLK = 256
NRB = NROWS // RBLK
BB = 256
NBB = B // BB

_SC_INFO = plsc.get_sparse_core_info()
_NC = _SC_INFO.num_cores
_NS = _SC_INFO.num_subcores
_NW = _NC * _NS
_CHUNK = 128
_NCH = NROWS // _CHUNK
_CH_PER_W = _NCH // _NW
_ROWS_PER_W = NROWS // _NW


def _silu(x):
    return x * jax.nn.sigmoid(x)


def _router_body(h_ref, wk_ref, wv_ref, idxk_ref, idxv_ref):
    h = h_ref[...]

    def step(j, carry):
        mk, ak, mv, av = carry
        start = pl.multiple_of(j * PCHUNK, PCHUNK)
        col = jax.lax.broadcasted_iota(jnp.int32, (RBLK, PCHUNK), 1) + j * PCHUNK
        valid = col < POOL

        def one(w_ref, m, a):
            logits = jnp.dot(h, w_ref[:, pl.ds(start, PCHUNK)],
                             preferred_element_type=jnp.float32)
            logits = jnp.where(valid, logits, -jnp.inf)
            m_c = jnp.max(logits, axis=1, keepdims=True)
            cand = jnp.where(logits == m_c, col, jnp.int32(2**30))
            a_c = jnp.min(cand, axis=1, keepdims=True)
            upd = m_c > m
            return jnp.where(upd, m_c, m), jnp.where(upd, a_c, a)

        mk, ak = one(wk_ref, mk, ak)
        mv, av = one(wv_ref, mv, av)
        return mk, ak, mv, av

    neg = jnp.full((RBLK, 1), -jnp.inf, jnp.float32)
    zero = jnp.zeros((RBLK, 1), jnp.int32)
    _, ak, _, av = jax.lax.fori_loop(0, NCHUNK, step, (neg, zero, neg, zero))
    idxk_ref[0, 0, :] = ak[:, 0]
    idxv_ref[0, 0, :] = av[:, 0]


def _attn_body(t_ref, h_ref, gk_ref, gv_ref, wq_ref, wk_ref, wv_ref, wo_ref,
               o_ref):
    q = _silu(jnp.dot(t_ref[...], wq_ref[...],
                      preferred_element_type=jnp.float32))
    hist = h_ref[...].reshape(BB * S, D)
    uk = _silu(_silu(jnp.dot(hist, wk_ref[...],
                             preferred_element_type=jnp.float32)))
    uv = _silu(_silu(jnp.dot(hist, wv_ref[...],
                             preferred_element_type=jnp.float32)))
    kfull = jnp.concatenate([_silu(gk_ref[...][:, :GLOB]), uk], axis=-1)
    vfull = jnp.concatenate([_silu(gv_ref[...][:, :GLOB]), uv], axis=-1)
    k3 = kfull.reshape(BB, S, D)
    v3 = vfull.reshape(BB, S, D)

    scale = DH ** 0.5
    outs = []
    for hh in range(H):
        sl = slice(hh * DH, (hh + 1) * DH)
        qh = q[:, sl]
        kh = k3[:, :, sl]
        s = jnp.sum(qh[:, None, :] * kh, axis=-1) / scale
        s = s - jnp.max(s, axis=1, keepdims=True)
        e = jnp.exp(s)
        a = e / jnp.sum(e, axis=1, keepdims=True)
        vh = v3[:, :, sl]
        outs.append(jnp.sum(a[:, :, None] * vh, axis=1))
    out = jnp.concatenate(outs, axis=-1)
    o_ref[...] = jnp.dot(out, wo_ref[...], preferred_element_type=jnp.float32)


def _sc_gather(kpool_hbm, vpool_hbm, idxk_hbm, idxv_hbm, outk_hbm, outv_hbm,
               idx_v, rows_v, sem):
    wid = lax.axis_index("s") * _NC + lax.axis_index("c")
    base_chunk = wid * _CH_PER_W
    base_row = wid * _ROWS_PER_W
    for pool_hbm, idx_hbm, out_hbm in ((kpool_hbm, idxk_hbm, outk_hbm),
                                       (vpool_hbm, idxv_hbm, outv_hbm)):
        pltpu.sync_copy(idx_hbm.at[pl.ds(base_chunk, _CH_PER_W)], idx_v)
        copies = [
            pltpu.async_copy(pool_hbm.at[idx_v.at[j]],
                             rows_v.at[pl.ds(j * _CHUNK, _CHUNK)], sem)
            for j in range(_CH_PER_W)
        ]
        for c in copies:
            c.wait()
        pltpu.sync_copy(rows_v, out_hbm.at[pl.ds(base_row, _ROWS_PER_W)])


def kernel(target_item, history_sequence, W_q, W_k, router_K_w, global_K_pool,
           W_v, router_V_w, global_V_pool, W_o):
    hist2d = history_sequence.reshape(NROWS, D)
    wk_pad = jnp.pad(router_K_w, ((0, 0), (0, PPAD - POOL)))
    wv_pad = jnp.pad(router_V_w, ((0, 0), (0, PPAD - POOL)))

    idxk3, idxv3 = pl.pallas_call(
        _router_body,
        grid=(NRB,),
        in_specs=[
            pl.BlockSpec((RBLK, D), lambda i: (i, 0)),
            pl.BlockSpec((D, PPAD), lambda i: (0, 0)),
            pl.BlockSpec((D, PPAD), lambda i: (0, 0)),
        ],
        out_specs=[
            pl.BlockSpec((1, 1, RBLK), lambda i: (i, 0, 0)),
            pl.BlockSpec((1, 1, RBLK), lambda i: (i, 0, 0)),
        ],
        out_shape=[
            jax.ShapeDtypeStruct((NRB, 1, RBLK), jnp.int32),
            jax.ShapeDtypeStruct((NRB, 1, RBLK), jnp.int32),
        ],
    )(hist2d, wk_pad, wv_pad)

    idxk2d = idxk3.reshape(_NCH, _CHUNK)
    idxv2d = idxv3.reshape(_NCH, _CHUNK)
    kpool_pad = jnp.pad(global_K_pool, ((0, 0), (0, D - GLOB)))
    vpool_pad = jnp.pad(global_V_pool, ((0, 0), (0, D - GLOB)))

    gather = functools.partial(
        pl.kernel,
        mesh=plsc.VectorSubcoreMesh(core_axis_name="c", subcore_axis_name="s"),
        out_type=(
            jax.ShapeDtypeStruct((NROWS, D), jnp.float32),
            jax.ShapeDtypeStruct((NROWS, D), jnp.float32),
        ),
        scratch_types=[
            pltpu.VMEM((_CH_PER_W, _CHUNK), jnp.int32),
            pltpu.VMEM((_ROWS_PER_W, D), jnp.float32),
            pltpu.SemaphoreType.DMA,
        ],
    )(_sc_gather)
    gk, gv = gather(kpool_pad, vpool_pad, idxk2d, idxv2d)

    out = pl.pallas_call(
        _attn_body,
        grid=(NBB,),
        in_specs=[
            pl.BlockSpec((BB, D), lambda i: (i, 0)),
            pl.BlockSpec((BB, S, D), lambda i: (i, 0, 0)),
            pl.BlockSpec((BB * S, D), lambda i: (i, 0)),
            pl.BlockSpec((BB * S, D), lambda i: (i, 0)),
            pl.BlockSpec((D, D), lambda i: (0, 0)),
            pl.BlockSpec((D, USR), lambda i: (0, 0)),
            pl.BlockSpec((D, USR), lambda i: (0, 0)),
            pl.BlockSpec((D, D), lambda i: (0, 0)),
        ],
        out_specs=pl.BlockSpec((BB, D), lambda i: (i, 0)),
        out_shape=jax.ShapeDtypeStruct((B, D), jnp.float32),
    )(target_item, history_sequence, gk, gv, W_q, W_k, W_v, W_o)
    return out


# speedup vs baseline: 1.0428x; 1.0428x over previous
import functools

import jax
import jax.numpy as jnp
from jax import lax
from jax.experimental import pallas as pl
from jax.experimental.pallas import tpu as pltpu
from jax.experimental.pallas import tpu_sc as plsc

B, S, D = 1024, 20, 64
H = 4
DH = D // H
USR = 8
GLOB = D - USR
POOL = 10000

NROWS = B * S
PPAD = 10240
PCHUNK = 1024
NCHUNK = PPAD // PCHUNK
RBLK = 256
NRB = NROWS // RBLK
BB = 256
NBB = B // BB

_NC = 2
_NS = 16
_NW = _NC * _NS
_CHUNK = 128
_NCH = NROWS // _CHUNK
_CH_PER_W = _NCH // _NW
_ROWS_PER_W = NROWS // _NW


def _silu(x):
    return x * jax.nn.sigmoid(x)


def _router_body(h_ref, wk_ref, wv_ref, idxk_ref, idxv_ref):
    h = h_ref[...].astype(jnp.bfloat16)

    def step(j, carry):
        mk, ak, mv, av = carry
        start = pl.multiple_of(j * PCHUNK, PCHUNK)
        col = jax.lax.broadcasted_iota(jnp.int32, (RBLK, PCHUNK), 1) + j * PCHUNK
        valid = col < POOL

        def one(w_ref, m, a):
            logits = jnp.dot(h, w_ref[:, pl.ds(start, PCHUNK)],
                             preferred_element_type=jnp.float32)
            logits = jnp.where(valid, logits, -jnp.inf)
            m_c = jnp.max(logits, axis=1, keepdims=True)
            cand = jnp.where(logits == m_c, col, jnp.int32(2**30))
            a_c = jnp.min(cand, axis=1, keepdims=True)
            upd = m_c > m
            return jnp.where(upd, m_c, m), jnp.where(upd, a_c, a)

        mk, ak = one(wk_ref, mk, ak)
        mv, av = one(wv_ref, mv, av)
        return mk, ak, mv, av

    neg = jnp.full((RBLK, 1), -jnp.inf, jnp.float32)
    zero = jnp.zeros((RBLK, 1), jnp.int32)
    _, ak, _, av = jax.lax.fori_loop(0, NCHUNK, step, (neg, zero, neg, zero))
    idxk_ref[0, 0, :] = ak[:, 0]
    idxv_ref[0, 0, :] = av[:, 0]


def _attn_body(t_ref, h_ref, gk_ref, gv_ref, wq_ref, wk_ref, wv_ref, wo_ref,
               o_ref):
    q = _silu(jnp.dot(t_ref[...], wq_ref[...],
                      preferred_element_type=jnp.float32))
    hist = h_ref[...].reshape(BB * S, D)
    uk = _silu(_silu(jnp.dot(hist, wk_ref[...],
                             preferred_element_type=jnp.float32)))
    uv = _silu(_silu(jnp.dot(hist, wv_ref[...],
                             preferred_element_type=jnp.float32)))
    kfull = jnp.concatenate([_silu(gk_ref[...][:, :GLOB]), uk], axis=-1)
    vfull = jnp.concatenate([_silu(gv_ref[...][:, :GLOB]), uv], axis=-1)
    k3 = kfull.reshape(BB, S, D)
    v3 = vfull.reshape(BB, S, D)

    scale = DH ** 0.5
    outs = []
    for hh in range(H):
        sl = slice(hh * DH, (hh + 1) * DH)
        qh = q[:, sl]
        kh = k3[:, :, sl]
        s = jnp.sum(qh[:, None, :] * kh, axis=-1) / scale
        s = s - jnp.max(s, axis=1, keepdims=True)
        e = jnp.exp(s)
        a = e / jnp.sum(e, axis=1, keepdims=True)
        vh = v3[:, :, sl]
        outs.append(jnp.sum(a[:, :, None] * vh, axis=1))
    out = jnp.concatenate(outs, axis=-1)
    o_ref[...] = jnp.dot(out, wo_ref[...], preferred_element_type=jnp.float32)


def _sc_gather(kpool_hbm, vpool_hbm, idxk_hbm, idxv_hbm, outk_hbm, outv_hbm,
               idx_v, rows_v, sem):
    wid = lax.axis_index("s") * _NC + lax.axis_index("c")
    base_row = wid * _ROWS_PER_W
    for pool_hbm, idx_hbm, out_hbm in ((kpool_hbm, idxk_hbm, outk_hbm),
                                       (vpool_hbm, idxv_hbm, outv_hbm)):
        pltpu.sync_copy(idx_hbm.at[pl.ds(base_row, _ROWS_PER_W)], idx_v)
        copies = [
            pltpu.async_copy(pool_hbm.at[idx_v.at[pl.ds(j * _CHUNK, _CHUNK)]],
                             rows_v.at[pl.ds(j * _CHUNK, _CHUNK)], sem)
            for j in range(_CH_PER_W)
        ]
        for c in copies:
            c.wait()
        pltpu.sync_copy(rows_v, out_hbm.at[pl.ds(base_row, _ROWS_PER_W)])


def kernel(target_item, history_sequence, W_q, W_k, router_K_w, global_K_pool,
           W_v, router_V_w, global_V_pool, W_o):
    hist2d = history_sequence.reshape(NROWS, D)

    pad = ((0, 0), (0, PPAD - POOL))
    wk_pad = jnp.pad(router_K_w.astype(jnp.bfloat16), pad)
    wv_pad = jnp.pad(router_V_w.astype(jnp.bfloat16), pad)

    wspec = pl.BlockSpec((D, PPAD), lambda i: (0, 0))
    idxk3, idxv3 = pl.pallas_call(
        _router_body,
        grid=(NRB,),
        in_specs=[
            pl.BlockSpec((RBLK, D), lambda i: (i, 0)),
            wspec, wspec,
        ],
        out_specs=[
            pl.BlockSpec((1, 1, RBLK), lambda i: (i, 0, 0)),
            pl.BlockSpec((1, 1, RBLK), lambda i: (i, 0, 0)),
        ],
        out_shape=[
            jax.ShapeDtypeStruct((NRB, 1, RBLK), jnp.int32),
            jax.ShapeDtypeStruct((NRB, 1, RBLK), jnp.int32),
        ],
    )(hist2d, wk_pad, wv_pad)

    idxk1d = idxk3.reshape(NROWS)
    idxv1d = idxv3.reshape(NROWS)
    kpool_pad = jnp.pad(global_K_pool, ((0, 0), (0, D - GLOB)))
    vpool_pad = jnp.pad(global_V_pool, ((0, 0), (0, D - GLOB)))

    gather = functools.partial(
        pl.kernel,
        mesh=plsc.VectorSubcoreMesh(core_axis_name="c", subcore_axis_name="s"),
        compiler_params=pltpu.CompilerParams(use_tc_tiling_on_sc=False),
        out_type=(
            jax.ShapeDtypeStruct((NROWS, D), jnp.float32),
            jax.ShapeDtypeStruct((NROWS, D), jnp.float32),
        ),
        scratch_types=[
            pltpu.VMEM((_ROWS_PER_W,), jnp.int32),
            pltpu.VMEM((_ROWS_PER_W, D), jnp.float32),
            pltpu.SemaphoreType.DMA,
        ],
    )(_sc_gather)
    gk, gv = gather(kpool_pad, vpool_pad, idxk1d, idxv1d)

    out = pl.pallas_call(
        _attn_body,
        grid=(NBB,),
        in_specs=[
            pl.BlockSpec((BB, D), lambda i: (i, 0)),
            pl.BlockSpec((BB, S, D), lambda i: (i, 0, 0)),
            pl.BlockSpec((BB * S, D), lambda i: (i, 0)),
            pl.BlockSpec((BB * S, D), lambda i: (i, 0)),
            pl.BlockSpec((D, D), lambda i: (0, 0)),
            pl.BlockSpec((D, USR), lambda i: (0, 0)),
            pl.BlockSpec((D, USR), lambda i: (0, 0)),
            pl.BlockSpec((D, D), lambda i: (0, 0)),
        ],
        out_specs=pl.BlockSpec((BB, D), lambda i: (i, 0)),
        out_shape=jax.ShapeDtypeStruct((B, D), jnp.float32),
    )(target_item, history_sequence, gk, gv, W_q, W_k, W_v, W_o)
    return out


# speedup vs baseline: 1.4068x; 1.3491x over previous
import functools

import jax
import jax.numpy as jnp
from jax import lax
from jax.experimental import pallas as pl
from jax.experimental.pallas import tpu as pltpu
from jax.experimental.pallas import tpu_sc as plsc

B, S, D = 1024, 20, 64
H = 4
DH = D // H
USR = 8
GLOB = D - USR
POOL = 10000

NROWS = B * S
PPAD = 10240
PCHUNK = 1024
NCHUNK = PPAD // PCHUNK
RBLK = 256
NRB = NROWS // RBLK
BB = 256
NBB = B // BB

_NC = 2
_NS = 16
_NW = _NC * _NS
_CHUNK = 128
_NCH = NROWS // _CHUNK
_CH_PER_W = _NCH // _NW
_ROWS_PER_W = NROWS // _NW


def _silu(x):
    return x * jax.nn.sigmoid(x)


def _router_body(h_ref, wk_ref, wv_ref, idxk_ref, idxv_ref):
    h = h_ref[...].astype(jnp.bfloat16)
    col0 = jax.lax.broadcasted_iota(jnp.int32, (RBLK, PCHUNK), 1)

    def one(w_ref, j, m, a):
        logits = jnp.dot(h, w_ref[:, j * PCHUNK:(j + 1) * PCHUNK],
                         preferred_element_type=jnp.float32)
        if (j + 1) * PCHUNK > POOL:
            logits = jnp.where(col0 + j * PCHUNK < POOL, logits, -jnp.inf)
        m_c = jnp.max(logits, axis=1, keepdims=True)
        cand = jnp.where(logits == m_c, col0 + j * PCHUNK, jnp.int32(2**30))
        a_c = jnp.min(cand, axis=1, keepdims=True)
        upd = m_c > m
        return jnp.where(upd, m_c, m), jnp.where(upd, a_c, a)

    neg = jnp.full((RBLK, 1), -jnp.inf, jnp.float32)
    zero = jnp.zeros((RBLK, 1), jnp.int32)
    mk, ak, mv, av = neg, zero, neg, zero
    for j in range(NCHUNK):
        mk, ak = one(wk_ref, j, mk, ak)
        mv, av = one(wv_ref, j, mv, av)
    idxk_ref[0, 0, :] = ak[:, 0]
    idxv_ref[0, 0, :] = av[:, 0]


def _attn_body(t_ref, h_ref, gk_ref, gv_ref, wq_ref, wk_ref, wv_ref, wo_ref,
               o_ref):
    q = _silu(jnp.dot(t_ref[...], wq_ref[...],
                      preferred_element_type=jnp.float32))
    hist = h_ref[...].reshape(BB * S, D)
    uk = _silu(_silu(jnp.dot(hist, wk_ref[...],
                             preferred_element_type=jnp.float32)))
    uv = _silu(_silu(jnp.dot(hist, wv_ref[...],
                             preferred_element_type=jnp.float32)))
    kfull = jnp.concatenate([_silu(gk_ref[...][:, :GLOB]), uk], axis=-1)
    vfull = jnp.concatenate([_silu(gv_ref[...][:, :GLOB]), uv], axis=-1)
    k3 = kfull.reshape(BB, S, D)
    v3 = vfull.reshape(BB, S, D)

    scale = DH ** 0.5
    outs = []
    for hh in range(H):
        sl = slice(hh * DH, (hh + 1) * DH)
        qh = q[:, sl]
        kh = k3[:, :, sl]
        s = jnp.sum(qh[:, None, :] * kh, axis=-1) / scale
        s = s - jnp.max(s, axis=1, keepdims=True)
        e = jnp.exp(s)
        a = e / jnp.sum(e, axis=1, keepdims=True)
        vh = v3[:, :, sl]
        outs.append(jnp.sum(a[:, :, None] * vh, axis=1))
    out = jnp.concatenate(outs, axis=-1)
    o_ref[...] = jnp.dot(out, wo_ref[...], preferred_element_type=jnp.float32)


def _sc_gather(kpool_hbm, vpool_hbm, idxk_hbm, idxv_hbm, outk_hbm, outv_hbm,
               idx_v, rows_v, sem):
    wid = lax.axis_index("s") * _NC + lax.axis_index("c")
    base_row = wid * _ROWS_PER_W
    for pool_hbm, idx_hbm, out_hbm in ((kpool_hbm, idxk_hbm, outk_hbm),
                                       (vpool_hbm, idxv_hbm, outv_hbm)):
        pltpu.sync_copy(idx_hbm.at[pl.ds(base_row, _ROWS_PER_W)], idx_v)
        copies = [
            pltpu.async_copy(pool_hbm.at[idx_v.at[pl.ds(j * _CHUNK, _CHUNK)]],
                             rows_v.at[pl.ds(j * _CHUNK, _CHUNK)], sem)
            for j in range(_CH_PER_W)
        ]
        for c in copies:
            c.wait()
        pltpu.sync_copy(rows_v, out_hbm.at[pl.ds(base_row, _ROWS_PER_W)])


def kernel(target_item, history_sequence, W_q, W_k, router_K_w, global_K_pool,
           W_v, router_V_w, global_V_pool, W_o):
    hist2d = history_sequence.reshape(NROWS, D)

    pad = ((0, 0), (0, PPAD - POOL))
    wk_pad = jnp.pad(router_K_w.astype(jnp.bfloat16), pad)
    wv_pad = jnp.pad(router_V_w.astype(jnp.bfloat16), pad)

    wspec = pl.BlockSpec((D, PPAD), lambda i: (0, 0))
    idxk3, idxv3 = pl.pallas_call(
        _router_body,
        grid=(NRB,),
        in_specs=[
            pl.BlockSpec((RBLK, D), lambda i: (i, 0)),
            wspec, wspec,
        ],
        out_specs=[
            pl.BlockSpec((1, 1, RBLK), lambda i: (i, 0, 0)),
            pl.BlockSpec((1, 1, RBLK), lambda i: (i, 0, 0)),
        ],
        out_shape=[
            jax.ShapeDtypeStruct((NRB, 1, RBLK), jnp.int32),
            jax.ShapeDtypeStruct((NRB, 1, RBLK), jnp.int32),
        ],
    )(hist2d, wk_pad, wv_pad)

    idxk1d = idxk3.reshape(NROWS)
    idxv1d = idxv3.reshape(NROWS)
    kpool_pad = jnp.pad(global_K_pool, ((0, 0), (0, D - GLOB)))
    vpool_pad = jnp.pad(global_V_pool, ((0, 0), (0, D - GLOB)))

    gather = functools.partial(
        pl.kernel,
        mesh=plsc.VectorSubcoreMesh(core_axis_name="c", subcore_axis_name="s"),
        compiler_params=pltpu.CompilerParams(use_tc_tiling_on_sc=False),
        out_type=(
            jax.ShapeDtypeStruct((NROWS, D), jnp.float32),
            jax.ShapeDtypeStruct((NROWS, D), jnp.float32),
        ),
        scratch_types=[
            pltpu.VMEM((_ROWS_PER_W,), jnp.int32),
            pltpu.VMEM((_ROWS_PER_W, D), jnp.float32),
            pltpu.SemaphoreType.DMA,
        ],
    )(_sc_gather)
    gk, gv = gather(kpool_pad, vpool_pad, idxk1d, idxv1d)

    out = pl.pallas_call(
        _attn_body,
        grid=(NBB,),
        in_specs=[
            pl.BlockSpec((BB, D), lambda i: (i, 0)),
            pl.BlockSpec((BB, S, D), lambda i: (i, 0, 0)),
            pl.BlockSpec((BB * S, D), lambda i: (i, 0)),
            pl.BlockSpec((BB * S, D), lambda i: (i, 0)),
            pl.BlockSpec((D, D), lambda i: (0, 0)),
            pl.BlockSpec((D, USR), lambda i: (0, 0)),
            pl.BlockSpec((D, USR), lambda i: (0, 0)),
            pl.BlockSpec((D, D), lambda i: (0, 0)),
        ],
        out_specs=pl.BlockSpec((BB, D), lambda i: (i, 0)),
        out_shape=jax.ShapeDtypeStruct((B, D), jnp.float32),
    )(target_item, history_sequence, gk, gv, W_q, W_k, W_v, W_o)
    return out
